# baseline (speedup 1.0000x reference)
import functools

import jax
import jax.numpy as jnp
from jax import lax
from jax.experimental import pallas as pl
from jax.experimental.pallas import tpu as pltpu
from jax.experimental.pallas import tpu_sc as plsc

D = 2048
H = 4096
E = 8
K = 2
N = 2048
ENTROPY_WEIGHT = 0.1
NT = 8
TBLK = N // NT
HC = H // NT
_BIG = 1 << 20
_HIGH = jax.lax.Precision.HIGHEST


def _probsT_body(x8_ref, gw_ref, gb_ref, pT_ref):
    lt = jax.lax.dot_general(
        gw_ref[...], x8_ref[...], (((1,), (1,)), ((), ())),
        preferred_element_type=jnp.float32, precision=_HIGH,
    ) + gb_ref[...]
    m = jnp.max(lt, axis=0, keepdims=True)
    ex = jnp.exp(lt - m)
    p = ex / jnp.sum(ex, axis=0, keepdims=True)
    pT_ref[...] = jnp.concatenate(
        [p, jnp.full((E, 8), -1.0, jnp.float32)], axis=1)


_SC_MESH = plsc.VectorSubcoreMesh(core_axis_name="c", subcore_axis_name="s")


def _sc_lane_gather(v, idx):
    return lax.gather(
        v, idx[:, None],
        lax.GatherDimensionNumbers(
            offset_dims=(), collapsed_slice_dims=(0,), start_index_map=(0,)),
        (1,), mode=lax.GatherScatterMode.PROMISE_IN_BOUNDS)


@functools.partial(
    pl.kernel,
    out_type=[
        jax.ShapeDtypeStruct((K, 16), jnp.float32),
        jax.ShapeDtypeStruct((K, 16), jnp.int32),
    ],
    mesh=_SC_MESH,
    scratch_types=[
        pltpu.VMEM((E, 16), jnp.float32),
        pltpu.VMEM((K, 16), jnp.float32),
        pltpu.VMEM((K, 16), jnp.int32),
    ],
)
def _sc_router(pT_hbm, coef_hbm, esel_hbm, p_v, coef_v, esel_v):
    wid = lax.axis_index("s") * 2 + lax.axis_index("c")

    @pl.when(wid == 0)
    def _():
        pltpu.sync_copy(pT_hbm, p_v)
        lane = lax.iota(jnp.int32, 16)
        valid = lane < 8
        rows = [p_v[e, :] for e in range(E)]
        bv1 = rows[0]
        bi1 = jnp.zeros((16,), jnp.int32)
        for e in range(1, E):
            upd = rows[e] > bv1
            bv1 = jnp.where(upd, rows[e], bv1)
            bi1 = jnp.where(upd, e, bi1)
        bv2 = jnp.full((16,), -2.0, jnp.float32)
        bi2 = jnp.zeros((16,), jnp.int32)
        for e in range(E):
            pe = jnp.where(bi1 == e, -2.0, rows[e])
            upd = pe > bv2
            bv2 = jnp.where(upd, pe, bv2)
            bi2 = jnp.where(upd, e, bi2)
        for k, (bv, bi) in enumerate(((bv1, bi1), (bv2, bi2))):
            eq = jnp.logical_and(bi == lane, valid)
            coef_v[k, :] = jnp.where(eq, bv, 0.0)
            key = jnp.where(eq, lane, 16)
            for shift in (1, 2, 4, 8):
                idx = jnp.bitwise_and(lane + shift, 15)
                key = jnp.minimum(key, _sc_lane_gather(key, idx))
            bi0 = _sc_lane_gather(bi, jnp.zeros((16,), jnp.int32))
            esel_v[k, :] = jnp.where(key < 16, key, bi0)
        pltpu.sync_copy(coef_v, coef_hbm)
        pltpu.sync_copy(esel_v, esel_hbm)


def _fused_body(esel_ref, x8_ref, coef_ref, w0a_ref, w0b_ref, w1a_ref,
                w1b_ref, b0_ref, b1_ref, xe_ref, gw_ref, gb_ref, out_ref,
                ent_ref, acc_ref):
    t = pl.program_id(0)

    for half, (wa, wb) in enumerate(((w0a_ref, w1a_ref), (w0b_ref, w1b_ref))):
        y0 = jax.lax.dot_general(
            x8_ref[...], wa[0], (((1,), (1,)), ((), ())),
            preferred_element_type=jnp.float32,
        )
        y1 = jax.lax.dot_general(
            x8_ref[...], wb[0], (((1,), (1,)), ((), ())),
            preferred_element_type=jnp.float32,
        )
        bsl = pl.ds(half * (HC // 2), HC // 2)
        y = ((y0 + b0_ref[0, :, bsl]) * coef_ref[0, :, :]
             + (y1 + b1_ref[0, :, bsl]) * coef_ref[1, :, :])
        acc_ref[:, pl.ds(t * HC + half * (HC // 2), HC // 2)] = y

    logits = jax.lax.dot_general(
        xe_ref[...], gw_ref[...], (((1,), (1,)), ((), ())),
        preferred_element_type=jnp.float32,
    ) + gb_ref[...]
    m = jnp.max(logits, axis=-1, keepdims=True)
    exl = jnp.exp(logits - m)
    p = exl / jnp.sum(exl, axis=-1, keepdims=True)
    ent = -jnp.sum(p * jnp.log(p + 1e-10))

    @pl.when(t == 0)
    def _():
        ent_ref[0, 0] = ent

    @pl.when(t != 0)
    def _():
        ent_ref[0, 0] += ent

    out_ref[...] = jnp.zeros_like(out_ref)

    @pl.when(t == NT - 1)
    def _():
        out_ref[0:8, :] = acc_ref[...]


def kernel(x, gate_W, gate_b, expert_W, expert_b):
    x_flat = x.reshape(N, D)
    x8 = x_flat[0:8]
    gb = gate_b.reshape(1, E)
    pT = pl.pallas_call(
        _probsT_body,
        in_specs=[
            pl.BlockSpec((8, D), lambda: (0, 0)),
            pl.BlockSpec((E, D), lambda: (0, 0)),
            pl.BlockSpec((E, 1), lambda: (0, 0)),
        ],
        out_specs=pl.BlockSpec((E, 16), lambda: (0, 0)),
        out_shape=jax.ShapeDtypeStruct((E, 16), jnp.float32),
    )(x8, gate_W, gate_b.reshape(E, 1))
    coef16, esel16 = _sc_router(pT)
    coef = coef16[:, 0:8].reshape(K, 8, 1)
    esel = esel16[:, 0]

    out, ent = pl.pallas_call(
        _fused_body,
        grid_spec=pltpu.PrefetchScalarGridSpec(
            num_scalar_prefetch=1,
            grid=(NT,),
            in_specs=[
                pl.BlockSpec((8, D), lambda t, s: (0, 0)),
                pl.BlockSpec((K, 8, 1), lambda t, s: (0, 0, 0)),
                pl.BlockSpec((1, HC // 2, D), lambda t, s: (s[0], 2 * t, 0)),
                pl.BlockSpec((1, HC // 2, D), lambda t, s: (s[0], 2 * t + 1, 0)),
                pl.BlockSpec((1, HC // 2, D), lambda t, s: (s[1], 2 * t, 0)),
                pl.BlockSpec((1, HC // 2, D), lambda t, s: (s[1], 2 * t + 1, 0)),
                pl.BlockSpec((1, 1, HC), lambda t, s: (s[0], 0, t)),
                pl.BlockSpec((1, 1, HC), lambda t, s: (s[1], 0, t)),
                pl.BlockSpec((TBLK, D), lambda t, s: (NT - 1 - t, 0)),
                pl.BlockSpec((E, D), lambda t, s: (0, 0)),
                pl.BlockSpec((1, E), lambda t, s: (0, 0)),
            ],
            out_specs=[
                pl.BlockSpec((TBLK, H), lambda t, s: (NT - 1 - t, 0)),
                pl.BlockSpec(memory_space=pltpu.SMEM),
            ],
            scratch_shapes=[pltpu.VMEM((8, H), jnp.float32)],
        ),
        out_shape=[
            jax.ShapeDtypeStruct((N, H), jnp.float32),
            jax.ShapeDtypeStruct((1, 1), jnp.float32),
        ],
    )(esel, x8, coef, expert_W, expert_W, expert_W, expert_W,
      expert_b.reshape(E, 1, H), expert_b.reshape(E, 1, H), x_flat,
      gate_W, gb)

    loss = ENTROPY_WEIGHT * ent[0, 0] / N
    return out.reshape(1, N, H), loss


# speedup vs baseline: 1.0387x; 1.0387x over previous
import functools

import jax
import jax.numpy as jnp
from jax import lax
from jax.experimental import pallas as pl
from jax.experimental.pallas import tpu as pltpu
from jax.experimental.pallas import tpu_sc as plsc

D = 2048
H = 4096
E = 8
K = 2
N = 2048
ENTROPY_WEIGHT = 0.1
NT = 8
TBLK = N // NT
HC = H // NT
_BIG = 1 << 20
_HIGH = jax.lax.Precision.HIGHEST


def _probsT_body(x8_ref, gw_ref, gb_ref, pT_ref):
    lt = jax.lax.dot_general(
        gw_ref[...], x8_ref[...], (((1,), (1,)), ((), ())),
        preferred_element_type=jnp.float32, precision=_HIGH,
    ) + gb_ref[...]
    m = jnp.max(lt, axis=0, keepdims=True)
    ex = jnp.exp(lt - m)
    p = ex / jnp.sum(ex, axis=0, keepdims=True)
    pT_ref[...] = jnp.concatenate(
        [p, jnp.full((E, 8), -1.0, jnp.float32)], axis=1)


_SC_MESH = plsc.VectorSubcoreMesh(core_axis_name="c", subcore_axis_name="s")


def _sc_lane_gather(v, idx):
    return lax.gather(
        v, idx[:, None],
        lax.GatherDimensionNumbers(
            offset_dims=(), collapsed_slice_dims=(0,), start_index_map=(0,)),
        (1,), mode=lax.GatherScatterMode.PROMISE_IN_BOUNDS)


@functools.partial(
    pl.kernel,
    out_type=[
        jax.ShapeDtypeStruct((8, 16), jnp.float32),
        jax.ShapeDtypeStruct((K, 16), jnp.int32),
    ],
    mesh=_SC_MESH,
    scratch_types=[
        pltpu.VMEM((E, 16), jnp.float32),
        pltpu.VMEM((8, 16), jnp.float32),
        pltpu.VMEM((K, 16), jnp.int32),
    ],
)
def _sc_router(pT_hbm, coef_hbm, esel_hbm, p_v, coef_v, esel_v):
    wid = lax.axis_index("s") * 2 + lax.axis_index("c")

    @pl.when(wid == 0)
    def _():
        pltpu.sync_copy(pT_hbm, p_v)
        lane = lax.iota(jnp.int32, 16)
        valid = lane < 8
        rows = [p_v[e, :] for e in range(E)]
        bv1 = rows[0]
        bi1 = jnp.zeros((16,), jnp.int32)
        for e in range(1, E):
            upd = rows[e] > bv1
            bv1 = jnp.where(upd, rows[e], bv1)
            bi1 = jnp.where(upd, e, bi1)
        bv2 = jnp.full((16,), -2.0, jnp.float32)
        bi2 = jnp.zeros((16,), jnp.int32)
        for e in range(E):
            pe = jnp.where(bi1 == e, -2.0, rows[e])
            upd = pe > bv2
            bv2 = jnp.where(upd, pe, bv2)
            bi2 = jnp.where(upd, e, bi2)
        coef_vecs = []
        for k, (bv, bi) in enumerate(((bv1, bi1), (bv2, bi2))):
            eq = jnp.logical_and(bi == lane, valid)
            coef_vecs.append(jnp.where(eq, bv, 0.0))
            key = jnp.where(eq, lane, 16)
            for shift in (1, 2, 4, 8):
                idx = jnp.bitwise_and(lane + shift, 15)
                key = jnp.minimum(key, _sc_lane_gather(key, idx))
            bi0 = _sc_lane_gather(bi, jnp.zeros((16,), jnp.int32))
            esel_v[k, :] = jnp.where(key < 16, key, bi0)
        for t in range(8):
            tidx = jnp.full((16,), t, jnp.int32)
            g0 = _sc_lane_gather(coef_vecs[0], tidx)
            g1 = _sc_lane_gather(coef_vecs[1], tidx)
            coef_v[t, :] = jnp.where(lane == 0, g0,
                                     jnp.where(lane == 1, g1, 0.0))
        pltpu.sync_copy(coef_v, coef_hbm)
        pltpu.sync_copy(esel_v, esel_hbm)


def _fused_body(esel_ref, x8_ref, coef_ref, w0a_ref, w0b_ref, w1a_ref,
                w1b_ref, b0_ref, b1_ref, xe_ref, gw_ref, gb_ref, out_ref,
                ent_ref, acc_ref):
    t = pl.program_id(0)

    for half, (wa, wb) in enumerate(((w0a_ref, w1a_ref), (w0b_ref, w1b_ref))):
        y0 = jax.lax.dot_general(
            x8_ref[...], wa[0], (((1,), (1,)), ((), ())),
            preferred_element_type=jnp.float32,
        )
        y1 = jax.lax.dot_general(
            x8_ref[...], wb[0], (((1,), (1,)), ((), ())),
            preferred_element_type=jnp.float32,
        )
        bsl = pl.ds(half * (HC // 2), HC // 2)
        y = ((y0 + b0_ref[0, :, bsl]) * coef_ref[:, 0:1]
             + (y1 + b1_ref[0, :, bsl]) * coef_ref[:, 1:2])
        acc_ref[:, pl.ds(t * HC + half * (HC // 2), HC // 2)] = y

    logits = jax.lax.dot_general(
        xe_ref[...], gw_ref[...], (((1,), (1,)), ((), ())),
        preferred_element_type=jnp.float32,
    ) + gb_ref[...]
    m = jnp.max(logits, axis=-1, keepdims=True)
    exl = jnp.exp(logits - m)
    p = exl / jnp.sum(exl, axis=-1, keepdims=True)
    ent = -jnp.sum(p * jnp.log(p + 1e-10))

    @pl.when(t == 0)
    def _():
        ent_ref[0, 0] = ent

    @pl.when(t != 0)
    def _():
        ent_ref[0, 0] += ent

    out_ref[...] = jnp.zeros_like(out_ref)

    @pl.when(t == NT - 1)
    def _():
        out_ref[0:8, :] = acc_ref[...]


def kernel(x, gate_W, gate_b, expert_W, expert_b):
    x_flat = x.reshape(N, D)
    x8 = x_flat[0:8]
    gb = gate_b.reshape(1, E)
    pT = pl.pallas_call(
        _probsT_body,
        in_specs=[
            pl.BlockSpec((8, D), lambda: (0, 0)),
            pl.BlockSpec((E, D), lambda: (0, 0)),
            pl.BlockSpec((E, 1), lambda: (0, 0)),
        ],
        out_specs=pl.BlockSpec((E, 16), lambda: (0, 0)),
        out_shape=jax.ShapeDtypeStruct((E, 16), jnp.float32),
    )(x8, gate_W, gate_b.reshape(E, 1))
    coef, esel = _sc_router(pT)

    out, ent = pl.pallas_call(
        _fused_body,
        grid_spec=pltpu.PrefetchScalarGridSpec(
            num_scalar_prefetch=1,
            grid=(NT,),
            in_specs=[
                pl.BlockSpec((8, D), lambda t, s: (0, 0)),
                pl.BlockSpec((8, 16), lambda t, s: (0, 0)),
                pl.BlockSpec((1, HC // 2, D),
                             lambda t, s: (s[0, 0], 2 * t, 0)),
                pl.BlockSpec((1, HC // 2, D),
                             lambda t, s: (s[0, 0], 2 * t + 1, 0)),
                pl.BlockSpec((1, HC // 2, D),
                             lambda t, s: (s[1, 0], 2 * t, 0)),
                pl.BlockSpec((1, HC // 2, D),
                             lambda t, s: (s[1, 0], 2 * t + 1, 0)),
                pl.BlockSpec((1, 1, HC), lambda t, s: (s[0, 0], 0, t)),
                pl.BlockSpec((1, 1, HC), lambda t, s: (s[1, 0], 0, t)),
                pl.BlockSpec((TBLK, D), lambda t, s: (NT - 1 - t, 0)),
                pl.BlockSpec((E, D), lambda t, s: (0, 0)),
                pl.BlockSpec((1, E), lambda t, s: (0, 0)),
            ],
            out_specs=[
                pl.BlockSpec((TBLK, H), lambda t, s: (NT - 1 - t, 0)),
                pl.BlockSpec(memory_space=pltpu.SMEM),
            ],
            scratch_shapes=[pltpu.VMEM((8, H), jnp.float32)],
        ),
        out_shape=[
            jax.ShapeDtypeStruct((N, H), jnp.float32),
            jax.ShapeDtypeStruct((1, 1), jnp.float32),
        ],
    )(esel, x8, coef, expert_W, expert_W, expert_W, expert_W,
      expert_b.reshape(E, 1, H), expert_b.reshape(E, 1, H), x_flat,
      gate_W, gb)

    loss = ENTROPY_WEIGHT * ent[0, 0] / N
    return out.reshape(1, N, H), loss
